# baseline (speedup 1.0000x reference)
import functools

import jax
import jax.numpy as jnp
from jax import lax
from jax.experimental import pallas as pl
from jax.experimental.pallas import tpu as pltpu
from jax.experimental.pallas import tpu_sc as plsc

_VOCAB_ROWS = 1000001
_D = 32
_B = 16384
_MAX_VOCAB = 1000000.0
_ALPHA = 0.75

_NC = 2
_NS = 16
_NW = _NC * _NS
_BPW = _B // _NW
_CHUNK = 128
_NCHUNK = _BPW // _CHUNK
_GROUPS = _BPW // 16


def _sc_body(t_ind, c_ind, t_embT, c_embT, t_bias, c_bias, out_hbm,
             tidx_v, cidx_v, tbuf, cbuf, tb_v, cb_v, s_v, sem):
    wid = lax.axis_index("s") * _NC + lax.axis_index("c")

    pltpu.sync_copy(t_ind.at[wid], tidx_v)
    pltpu.sync_copy(c_ind.at[wid], cidx_v)

    bias_copies = []
    for j in range(_NCHUNK):
        r = pl.ds(j * _CHUNK, _CHUNK)
        bias_copies.append(pltpu.async_copy(t_bias.at[tidx_v.at[r]], tb_v.at[r], sem))
        bias_copies.append(pltpu.async_copy(c_bias.at[cidx_v.at[r]], cb_v.at[r], sem))

    def dfire(d, carry):
        for j in range(_NCHUNK):
            r = pl.ds(j * _CHUNK, _CHUNK)
            pltpu.async_copy(t_embT.at[d].at[tidx_v.at[r]], tbuf.at[d, r], sem)
            pltpu.async_copy(c_embT.at[d].at[cidx_v.at[r]], cbuf.at[d, r], sem)
        return carry

    lax.fori_loop(0, _D, dfire, 0)

    pltpu.make_async_copy(t_embT.at[:, pl.ds(0, _BPW)], tbuf, sem).wait()
    pltpu.make_async_copy(c_embT.at[:, pl.ds(0, _BPW)], cbuf, sem).wait()
    for c in bias_copies:
        c.wait()

    def group(g, carry):
        o = pl.ds(g * 16, 16)
        acc = tb_v[o] + cb_v[o]
        for d in range(_D):
            acc = acc + tbuf[d, o] * cbuf[d, o]
        s_v[o] = acc
        return carry

    lax.fori_loop(0, _GROUPS, group, 0)

    pltpu.sync_copy(s_v, out_hbm.at[pl.ds(wid * _BPW, _BPW)])


@functools.cache
def _make_sc_dot():
    @functools.partial(
        pl.kernel,
        mesh=plsc.VectorSubcoreMesh(core_axis_name="c", subcore_axis_name="s"),
        out_type=jax.ShapeDtypeStruct((_B,), jnp.float32),
        compiler_params=pltpu.CompilerParams(use_tc_tiling_on_sc=False),
        scratch_types=[
            pltpu.VMEM((_BPW,), jnp.int32),
            pltpu.VMEM((_BPW,), jnp.int32),
            pltpu.VMEM((_D, _BPW), jnp.float32),
            pltpu.VMEM((_D, _BPW), jnp.float32),
            pltpu.VMEM((_BPW,), jnp.float32),
            pltpu.VMEM((_BPW,), jnp.float32),
            pltpu.VMEM((_BPW,), jnp.float32),
            pltpu.SemaphoreType.DMA,
        ],
    )
    def _sc_dot(t_ind, c_ind, t_embT, c_embT, t_bias, c_bias, out_hbm, *scratch):
        _sc_body(t_ind, c_ind, t_embT, c_embT, t_bias, c_bias, out_hbm, *scratch)

    return _sc_dot


def _tc_tail_body(s_ref, co_ref, out_ref):
    s = s_ref[...]
    co = co_ref[...]
    w = jnp.minimum(1.0, jnp.power(co * (1.0 / _MAX_VOCAB), _ALPHA))
    diff = s - jnp.log(co + 1.0)
    out_ref[0, 0] = jnp.sum(w * diff * diff)


_tc_tail = pl.pallas_call(
    _tc_tail_body,
    out_shape=jax.ShapeDtypeStruct((1, 1), jnp.float32),
    out_specs=pl.BlockSpec(memory_space=pltpu.SMEM),
)


def kernel(target_ind, context_ind, co_occurs, target_embeddings,
           context_embeddings, target_biases, context_biases):
    tind = target_ind.astype(jnp.int32).reshape(_NW, _BPW)
    cind = context_ind.astype(jnp.int32).reshape(_NW, _BPW)
    s = _make_sc_dot()(tind, cind, target_embeddings.T, context_embeddings.T,
                       target_biases, context_biases)
    cost = _tc_tail(s.reshape(128, 128), co_occurs.astype(jnp.float32).reshape(128, 128))
    return cost[0, 0]


# speedup vs baseline: 9.4332x; 9.4332x over previous
import functools

import jax
import jax.numpy as jnp
from jax import lax
from jax.experimental import pallas as pl
from jax.experimental.pallas import tpu as pltpu
from jax.experimental.pallas import tpu_sc as plsc

_VOCAB_ROWS = 1000001
_D = 32
_B = 16384
_MAX_VOCAB = 1000000.0
_ALPHA = 0.75

_NC = 2
_NS = 16
_NW = _NC * _NS
_BPW = _B // _NW
_BLK = 128
_NBLK = _BPW // _BLK

_Q = 1 << 18
_PACK_COLS = 8192
_PACK_M = _Q // _PACK_COLS
_BIAS_ROWS = 7816


def _pack_body(t0, t1, t2, t3, out_ref):
    for u, tr in enumerate([t0, t1, t2, t3]):
        for k in range(_PACK_COLS // 512):
            out_ref[pl.ds(k * 512, 512), pl.ds(u * _D, _D)] = (
                tr[:, pl.ds(k * 512, 512)].T)


def _in_spec(u):
    if u < 3:
        return pl.BlockSpec((_D, _PACK_COLS), lambda m, u=u: (0, _PACK_M * u + m))
    return pl.BlockSpec(
        (_D, _PACK_COLS), lambda m: (0, _PACK_M * 3 + jnp.minimum(m, 26)))


_pack_call = pl.pallas_call(
    _pack_body,
    grid=(_PACK_M,),
    in_specs=[_in_spec(u) for u in range(4)],
    out_specs=pl.BlockSpec((_PACK_COLS, 128), lambda m: (m, 0)),
    out_shape=jax.ShapeDtypeStruct((_Q, 128), jnp.float32),
)


def _pack(tT):
    return _pack_call(tT, tT, tT, tT)


def _sc_body(t_ind, c_ind, t_pk, c_pk, t_b2, c_b2, out_hbm,
             tidx_v, cidx_v, tq_v, cq_v, tbq_v, cbq_v,
             trows, crows, tbr, cbr, s_v, sem):
    wid = lax.axis_index("s") * _NC + lax.axis_index("c")

    pltpu.sync_copy(t_ind.at[wid], tidx_v)
    pltpu.sync_copy(c_ind.at[wid], cidx_v)

    for j in range(_NBLK):
        for k in range(_BLK // 16):
            o = pl.ds(k * 16, 16)
            i = pl.ds(j * _BLK + k * 16, 16)
            tv = tidx_v[i]
            cv = cidx_v[i]
            tq_v[j, o] = tv & (_Q - 1)
            cq_v[j, o] = cv & (_Q - 1)
            tbq_v[j, o] = tv >> 7
            cbq_v[j, o] = cv >> 7

    lane = lax.iota(jnp.int32, 16)

    for b in range(_NBLK):
        base = b * _BLK
        cp1 = pltpu.async_copy(t_pk.at[tq_v.at[b]], trows, sem)
        cp2 = pltpu.async_copy(c_pk.at[cq_v.at[b]], crows, sem)
        cp3 = pltpu.async_copy(t_b2.at[tbq_v.at[b]], tbr, sem)
        cp4 = pltpu.async_copy(c_b2.at[cbq_v.at[b]], cbr, sem)
        cp1.wait()
        cp2.wait()
        cp3.wait()
        cp4.wait()

        def grp(g, carry2, base=base):
            o = pl.ds(base + g * 16, 16)
            rows = g * 16 + lane
            tcol = (tidx_v[o] >> 18) * _D
            ccol = (cidx_v[o] >> 18) * _D
            acc = (plsc.load_gather(tbr, [rows, tidx_v[o] & 127])
                   + plsc.load_gather(cbr, [rows, cidx_v[o] & 127]))
            for d in range(_D):
                acc = acc + (plsc.load_gather(trows, [rows, tcol + d])
                             * plsc.load_gather(crows, [rows, ccol + d]))
            s_v[o] = acc
            return carry2

        lax.fori_loop(0, _BLK // 16, grp, 0)

    pltpu.sync_copy(s_v, out_hbm.at[pl.ds(wid * _BPW, _BPW)])


@functools.cache
def _make_sc_dot():
    @functools.partial(
        pl.kernel,
        mesh=plsc.VectorSubcoreMesh(core_axis_name="c", subcore_axis_name="s"),
        out_type=jax.ShapeDtypeStruct((_B,), jnp.float32),
        compiler_params=pltpu.CompilerParams(
            needs_layout_passes=False, use_tc_tiling_on_sc=False),
        scratch_types=[
            pltpu.VMEM((_BPW,), jnp.int32),
            pltpu.VMEM((_BPW,), jnp.int32),
            pltpu.VMEM((_NBLK, _BLK), jnp.int32),
            pltpu.VMEM((_NBLK, _BLK), jnp.int32),
            pltpu.VMEM((_NBLK, _BLK), jnp.int32),
            pltpu.VMEM((_NBLK, _BLK), jnp.int32),
            pltpu.VMEM((_BLK, 128), jnp.float32),
            pltpu.VMEM((_BLK, 128), jnp.float32),
            pltpu.VMEM((_BLK, 128), jnp.float32),
            pltpu.VMEM((_BLK, 128), jnp.float32),
            pltpu.VMEM((_BPW,), jnp.float32),
            pltpu.SemaphoreType.DMA,
        ],
    )
    def _sc_dot(t_ind, c_ind, t_pk, c_pk, t_b2, c_b2, out_hbm, *scratch):
        _sc_body(t_ind, c_ind, t_pk, c_pk, t_b2, c_b2, out_hbm, *scratch)

    return _sc_dot


def _tc_tail_body(s_ref, co_ref, out_ref):
    s = s_ref[...]
    co = co_ref[...]
    w = jnp.minimum(1.0, jnp.power(co * (1.0 / _MAX_VOCAB), _ALPHA))
    diff = s - jnp.log(co + 1.0)
    out_ref[0, 0] = jnp.sum(w * diff * diff)


_tc_tail = pl.pallas_call(
    _tc_tail_body,
    out_shape=jax.ShapeDtypeStruct((1, 1), jnp.float32),
    out_specs=pl.BlockSpec(memory_space=pltpu.SMEM),
)


def _pad_bias(b):
    return jnp.pad(b, (0, _BIAS_ROWS * 128 - _VOCAB_ROWS)).reshape(_BIAS_ROWS, 128)


def kernel(target_ind, context_ind, co_occurs, target_embeddings,
           context_embeddings, target_biases, context_biases):
    tind = target_ind.astype(jnp.int32).reshape(_NW, _BPW)
    cind = context_ind.astype(jnp.int32).reshape(_NW, _BPW)
    t_pk = _pack(target_embeddings.T)
    c_pk = _pack(context_embeddings.T)
    s = _make_sc_dot()(tind, cind, t_pk, c_pk,
                       _pad_bias(target_biases), _pad_bias(context_biases))
    cost = _tc_tail(s.reshape(128, 128), co_occurs.astype(jnp.float32).reshape(128, 128))
    return cost[0, 0]


# speedup vs baseline: 16.8565x; 1.7869x over previous
import functools

import jax
import jax.numpy as jnp
from jax import lax
from jax.experimental import pallas as pl
from jax.experimental.pallas import tpu as pltpu
from jax.experimental.pallas import tpu_sc as plsc

_VOCAB_ROWS = 1000001
_D = 32
_B = 16384
_MAX_VOCAB = 1000000.0
_ALPHA = 0.75

_NC = 2
_NS = 16
_NW = _NC * _NS
_BPW = _B // _NW
_CHUNK = 128
_NCHUNK = _BPW // _CHUNK
_GROUPS = _BPW // 16

_DT_COLS = 8192
_DT_GRID = 123
_PLANE = _DT_GRID * _DT_COLS
_PB = _PLANE // 128


def _detile_body(t_ref, out_ref):
    out_ref[...] = t_ref[...].reshape(_D, _DT_COLS // 128, 128)


_detile = pl.pallas_call(
    _detile_body,
    grid=(_DT_GRID,),
    in_specs=[pl.BlockSpec((_D, _DT_COLS), lambda m: (0, m))],
    out_specs=pl.BlockSpec((_D, _DT_COLS // 128, 128), lambda m: (0, m, 0)),
    out_shape=jax.ShapeDtypeStruct((_D, _PB, 128), jnp.float32),
)


def _sc_body(t_ind, c_ind, t_lin, c_lin, t_bias, c_bias, out_hbm,
             tidx_v, cidx_v, tbuf, cbuf, tb_v, cb_v, s_v, sem):
    wid = lax.axis_index("s") * _NC + lax.axis_index("c")

    pltpu.sync_copy(t_ind.at[wid], tidx_v)
    pltpu.sync_copy(c_ind.at[wid], cidx_v)

    bias_copies = []
    for j in range(_NCHUNK):
        r = pl.ds(j * _CHUNK, _CHUNK)
        bias_copies.append(pltpu.async_copy(t_bias.at[tidx_v.at[r]], tb_v.at[r], sem))
        bias_copies.append(pltpu.async_copy(c_bias.at[cidx_v.at[r]], cb_v.at[r], sem))

    def dfire(d, carry):
        for j in range(_NCHUNK):
            r = pl.ds(j * _CHUNK, _CHUNK)
            pltpu.async_copy(t_lin.at[d].at[tidx_v.at[r]], tbuf.at[d, r], sem)
            pltpu.async_copy(c_lin.at[d].at[cidx_v.at[r]], cbuf.at[d, r], sem)
        return carry

    lax.fori_loop(0, _D, dfire, 0)

    pltpu.make_async_copy(t_lin.at[:, pl.ds(0, _BPW)], tbuf, sem).wait()
    pltpu.make_async_copy(c_lin.at[:, pl.ds(0, _BPW)], cbuf, sem).wait()
    for c in bias_copies:
        c.wait()

    def group(g, carry):
        o = pl.ds(g * 16, 16)
        acc = tb_v[o] + cb_v[o]
        for d in range(_D):
            acc = acc + tbuf[d, o] * cbuf[d, o]
        s_v[o] = acc
        return carry

    lax.fori_loop(0, _GROUPS, group, 0)

    pltpu.sync_copy(s_v, out_hbm.at[pl.ds(wid * _BPW, _BPW)])


@functools.cache
def _make_sc_dot():
    @functools.partial(
        pl.kernel,
        mesh=plsc.VectorSubcoreMesh(core_axis_name="c", subcore_axis_name="s"),
        out_type=jax.ShapeDtypeStruct((_B,), jnp.float32),
        compiler_params=pltpu.CompilerParams(use_tc_tiling_on_sc=False),
        scratch_types=[
            pltpu.VMEM((_BPW,), jnp.int32),
            pltpu.VMEM((_BPW,), jnp.int32),
            pltpu.VMEM((_D, _BPW), jnp.float32),
            pltpu.VMEM((_D, _BPW), jnp.float32),
            pltpu.VMEM((_BPW,), jnp.float32),
            pltpu.VMEM((_BPW,), jnp.float32),
            pltpu.VMEM((_BPW,), jnp.float32),
            pltpu.SemaphoreType.DMA,
        ],
    )
    def _sc_dot(t_ind, c_ind, t_lin, c_lin, t_bias, c_bias, out_hbm, *scratch):
        _sc_body(t_ind, c_ind, t_lin, c_lin, t_bias, c_bias, out_hbm, *scratch)

    return _sc_dot


def _tc_tail_body(s_ref, co_ref, out_ref):
    s = s_ref[...]
    co = co_ref[...]
    w = jnp.minimum(1.0, jnp.power(co * (1.0 / _MAX_VOCAB), _ALPHA))
    diff = s - jnp.log(co + 1.0)
    out_ref[0, 0] = jnp.sum(w * diff * diff)


_tc_tail = pl.pallas_call(
    _tc_tail_body,
    out_shape=jax.ShapeDtypeStruct((1, 1), jnp.float32),
    out_specs=pl.BlockSpec(memory_space=pltpu.SMEM),
)


def kernel(target_ind, context_ind, co_occurs, target_embeddings,
           context_embeddings, target_biases, context_biases):
    tind = target_ind.astype(jnp.int32).reshape(_NW, _BPW)
    cind = context_ind.astype(jnp.int32).reshape(_NW, _BPW)
    t_lin = _detile(target_embeddings.T).reshape(_D, _PLANE)
    c_lin = _detile(context_embeddings.T).reshape(_D, _PLANE)
    s = _make_sc_dot()(tind, cind, t_lin, c_lin,
                       target_biases, context_biases)
    cost = _tc_tail(s.reshape(128, 128), co_occurs.astype(jnp.float32).reshape(128, 128))
    return cost[0, 0]


# speedup vs baseline: 21.5980x; 1.2813x over previous
import functools

import jax
import jax.numpy as jnp
from jax import lax
from jax.experimental import pallas as pl
from jax.experimental.pallas import tpu as pltpu
from jax.experimental.pallas import tpu_sc as plsc

_VOCAB_ROWS = 1000001
_D = 32
_B = 16384
_MAX_VOCAB = 1000000.0
_ALPHA = 0.75

_NC = 2
_NS = 16
_NW = _NC * _NS
_BPW = _B // _NW
_CHUNK = 128
_NCHUNK = _BPW // _CHUNK
_GROUPS = _BPW // 16

_DT_COLS = 16384
_DT_GRID = 62
_PLANE = _DT_GRID * _DT_COLS
_PB = _PLANE // 128


def _detile_body(t_ref, out_ref):
    out_ref[...] = t_ref[...].reshape(_D, _DT_COLS // 128, 128)


_detile = pl.pallas_call(
    _detile_body,
    grid=(_DT_GRID,),
    in_specs=[pl.BlockSpec((_D, _DT_COLS), lambda m: (0, m))],
    out_specs=pl.BlockSpec((_D, _DT_COLS // 128, 128), lambda m: (0, m, 0)),
    out_shape=jax.ShapeDtypeStruct((_D, _PB, 128), jnp.float32),
)


def _sc_body(t_ind, c_ind, t_lin, c_lin, t_bias, c_bias, out_hbm,
             tidx_v, cidx_v, tbuf, cbuf, tb_v, cb_v, s_v, sem):
    wid = lax.axis_index("s") * _NC + lax.axis_index("c")

    pltpu.sync_copy(t_ind.at[wid], tidx_v)
    pltpu.sync_copy(c_ind.at[wid], cidx_v)

    bias_copies = []
    for j in range(_NCHUNK):
        r = pl.ds(j * _CHUNK, _CHUNK)
        bias_copies.append(pltpu.async_copy(t_bias.at[tidx_v.at[r]], tb_v.at[r], sem))
        bias_copies.append(pltpu.async_copy(c_bias.at[cidx_v.at[r]], cb_v.at[r], sem))

    def dfire(d, carry):
        for j in range(_NCHUNK):
            r = pl.ds(j * _CHUNK, _CHUNK)
            pltpu.async_copy(t_lin.at[d].at[tidx_v.at[r]], tbuf.at[d, r], sem)
            pltpu.async_copy(c_lin.at[d].at[cidx_v.at[r]], cbuf.at[d, r], sem)
        return carry

    lax.fori_loop(0, _D, dfire, 0)

    pltpu.make_async_copy(t_lin.at[:, pl.ds(0, _BPW)], tbuf, sem).wait()
    pltpu.make_async_copy(c_lin.at[:, pl.ds(0, _BPW)], cbuf, sem).wait()
    for c in bias_copies:
        c.wait()

    def group(g, carry):
        o = pl.ds(g * 16, 16)
        acc = tb_v[o] + cb_v[o]
        for d in range(_D):
            acc = acc + tbuf[d, o] * cbuf[d, o]
        s_v[o] = acc
        return carry

    lax.fori_loop(0, _GROUPS, group, 0)

    pltpu.sync_copy(s_v, out_hbm.at[pl.ds(wid * _BPW, _BPW)])


@functools.cache
def _make_sc_dot():
    @functools.partial(
        pl.kernel,
        mesh=plsc.VectorSubcoreMesh(core_axis_name="c", subcore_axis_name="s"),
        out_type=jax.ShapeDtypeStruct((_B,), jnp.float32),
        compiler_params=pltpu.CompilerParams(use_tc_tiling_on_sc=False),
        scratch_types=[
            pltpu.VMEM((_BPW,), jnp.int32),
            pltpu.VMEM((_BPW,), jnp.int32),
            pltpu.VMEM((_D, _BPW), jnp.float32),
            pltpu.VMEM((_D, _BPW), jnp.float32),
            pltpu.VMEM((_BPW,), jnp.float32),
            pltpu.VMEM((_BPW,), jnp.float32),
            pltpu.VMEM((_BPW,), jnp.float32),
            pltpu.SemaphoreType.DMA,
        ],
    )
    def _sc_dot(t_ind, c_ind, t_lin, c_lin, t_bias, c_bias, out_hbm, *scratch):
        _sc_body(t_ind, c_ind, t_lin, c_lin, t_bias, c_bias, out_hbm, *scratch)

    return _sc_dot


def _tc_tail_body(s_ref, co_ref, out_ref):
    s = s_ref[...]
    co = co_ref[...]
    w = jnp.minimum(1.0, jnp.power(co * (1.0 / _MAX_VOCAB), _ALPHA))
    diff = s - jnp.log(co + 1.0)
    out_ref[0, 0] = jnp.sum(w * diff * diff)


_tc_tail = pl.pallas_call(
    _tc_tail_body,
    out_shape=jax.ShapeDtypeStruct((1, 1), jnp.float32),
    out_specs=pl.BlockSpec(memory_space=pltpu.SMEM),
)


def kernel(target_ind, context_ind, co_occurs, target_embeddings,
           context_embeddings, target_biases, context_biases):
    tind = target_ind.astype(jnp.int32).reshape(_NW, _BPW)
    cind = context_ind.astype(jnp.int32).reshape(_NW, _BPW)
    t_lin = _detile(target_embeddings.T).reshape(_D, _PLANE)
    c_lin = _detile(context_embeddings.T).reshape(_D, _PLANE)
    s = _make_sc_dot()(tind, cind, t_lin, c_lin,
                       target_biases, context_biases)
    cost = _tc_tail(s.reshape(128, 128), co_occurs.astype(jnp.float32).reshape(128, 128))
    return cost[0, 0]


# speedup vs baseline: 23.4171x; 1.0842x over previous
import functools

import jax
import jax.numpy as jnp
from jax import lax
from jax.experimental import pallas as pl
from jax.experimental.pallas import tpu as pltpu
from jax.experimental.pallas import tpu_sc as plsc

_VOCAB_ROWS = 1000001
_D = 32
_B = 16384
_MAX_VOCAB = 1000000.0
_ALPHA = 0.75

_NC = 2
_NS = 16
_NW = _NC * _NS
_BPW = _B // _NW
_CHUNK = 128
_NCHUNK = _BPW // _CHUNK
_GROUPS = _BPW // 16

_DT_COLS = 32768
_DT_GRID = 31
_PLANE = _DT_GRID * _DT_COLS
_PB = _PLANE // 128


def _detile_body(t_ref, out_ref):
    out_ref[...] = t_ref[...].reshape(_D, _DT_COLS // 128, 128)


_detile = pl.pallas_call(
    _detile_body,
    grid=(_DT_GRID,),
    in_specs=[pl.BlockSpec((_D, _DT_COLS), lambda m: (0, m))],
    out_specs=pl.BlockSpec((_D, _DT_COLS // 128, 128), lambda m: (0, m, 0)),
    out_shape=jax.ShapeDtypeStruct((_D, _PB, 128), jnp.float32),
)


def _sc_body(t_ind, c_ind, t_lin, c_lin, t_bias, c_bias, out_hbm,
             tidx_v, cidx_v, tbuf, cbuf, tb_v, cb_v, s_v, sem):
    wid = lax.axis_index("s") * _NC + lax.axis_index("c")

    pltpu.sync_copy(t_ind.at[wid], tidx_v)
    pltpu.sync_copy(c_ind.at[wid], cidx_v)

    bias_copies = []
    for j in range(_NCHUNK):
        r = pl.ds(j * _CHUNK, _CHUNK)
        bias_copies.append(pltpu.async_copy(t_bias.at[tidx_v.at[r]], tb_v.at[r], sem))
        bias_copies.append(pltpu.async_copy(c_bias.at[cidx_v.at[r]], cb_v.at[r], sem))

    def dfire(d, carry):
        for j in range(_NCHUNK):
            r = pl.ds(j * _CHUNK, _CHUNK)
            pltpu.async_copy(t_lin.at[d].at[tidx_v.at[r]], tbuf.at[d, r], sem)
            pltpu.async_copy(c_lin.at[d].at[cidx_v.at[r]], cbuf.at[d, r], sem)
        return carry

    lax.fori_loop(0, _D, dfire, 0)

    pltpu.make_async_copy(t_lin.at[:, pl.ds(0, _BPW)], tbuf, sem).wait()
    pltpu.make_async_copy(c_lin.at[:, pl.ds(0, _BPW)], cbuf, sem).wait()
    for c in bias_copies:
        c.wait()

    def group(g, carry):
        o = pl.ds(g * 16, 16)
        acc = tb_v[o] + cb_v[o]
        for d in range(_D):
            acc = acc + tbuf[d, o] * cbuf[d, o]
        s_v[o] = acc
        return carry

    lax.fori_loop(0, _GROUPS, group, 0)

    pltpu.sync_copy(s_v, out_hbm.at[pl.ds(wid * _BPW, _BPW)])


@functools.cache
def _make_sc_dot():
    @functools.partial(
        pl.kernel,
        mesh=plsc.VectorSubcoreMesh(core_axis_name="c", subcore_axis_name="s"),
        out_type=jax.ShapeDtypeStruct((_B,), jnp.float32),
        compiler_params=pltpu.CompilerParams(use_tc_tiling_on_sc=False),
        scratch_types=[
            pltpu.VMEM((_BPW,), jnp.int32),
            pltpu.VMEM((_BPW,), jnp.int32),
            pltpu.VMEM((_D, _BPW), jnp.float32),
            pltpu.VMEM((_D, _BPW), jnp.float32),
            pltpu.VMEM((_BPW,), jnp.float32),
            pltpu.VMEM((_BPW,), jnp.float32),
            pltpu.VMEM((_BPW,), jnp.float32),
            pltpu.SemaphoreType.DMA,
        ],
    )
    def _sc_dot(t_ind, c_ind, t_lin, c_lin, t_bias, c_bias, out_hbm, *scratch):
        _sc_body(t_ind, c_ind, t_lin, c_lin, t_bias, c_bias, out_hbm, *scratch)

    return _sc_dot


def _tc_tail_body(s_ref, co_ref, out_ref):
    s = s_ref[...]
    co = co_ref[...]
    w = jnp.minimum(1.0, jnp.power(co * (1.0 / _MAX_VOCAB), _ALPHA))
    diff = s - jnp.log(co + 1.0)
    out_ref[0, 0] = jnp.sum(w * diff * diff)


_tc_tail = pl.pallas_call(
    _tc_tail_body,
    out_shape=jax.ShapeDtypeStruct((1, 1), jnp.float32),
    out_specs=pl.BlockSpec(memory_space=pltpu.SMEM),
)


def kernel(target_ind, context_ind, co_occurs, target_embeddings,
           context_embeddings, target_biases, context_biases):
    tind = target_ind.astype(jnp.int32).reshape(_NW, _BPW)
    cind = context_ind.astype(jnp.int32).reshape(_NW, _BPW)
    t_lin = _detile(target_embeddings.T).reshape(_D, _PLANE)
    c_lin = _detile(context_embeddings.T).reshape(_D, _PLANE)
    s = _make_sc_dot()(tind, cind, t_lin, c_lin,
                       target_biases, context_biases)
    cost = _tc_tail(s.reshape(128, 128), co_occurs.astype(jnp.float32).reshape(128, 128))
    return cost[0, 0]
